# baseline (speedup 1.0000x reference)
import functools

import jax
import jax.numpy as jnp
from jax.experimental import pallas as pl
from jax.experimental.pallas import tpu as pltpu

NUM_EXPERTS = 5
TOP_K = 2
EXPERT_LABELS = 2
HIDDEN = 768
EL = NUM_EXPERTS * EXPERT_LABELS
BB = 512
NSPLIT = 4
NB = 4096 // BB


def _moe_body(hs_hbm, wg1_ref, bg1_ref, wg2_ref, bg2_ref, wall_ref, ball_ref,
              final_ref, probs_ref, expert_ref, cls_buf, sems):
    i = pl.program_id(0)
    n = pl.num_programs(0)

    CH = BB // NSPLIT

    def cls_copy(slot, j):
        return pltpu.make_async_copy(
            hs_hbm.at[pl.ds(slot * BB + j * CH, CH), 0, :],
            cls_buf.at[slot, pl.ds(j * CH, CH), :],
            sems.at[slot, j],
        )

    @pl.when(i == 0)
    def _prime():
        for s in range(NB):
            for j in range(NSPLIT):
                cls_copy(s, j).start()

    for j in range(NSPLIT):
        cls_copy(i, j).wait()
    cls = cls_buf[i]

    h = jnp.maximum(
        jnp.dot(cls, wg1_ref[...], preferred_element_type=jnp.float32)
        + bg1_ref[...], 0.0)
    glog = jnp.dot(h, wg2_ref[...], preferred_element_type=jnp.float32) + bg2_ref[...]
    gmax = jnp.max(glog, axis=1, keepdims=True)
    ex = jnp.exp(glog - gmax)
    probs = ex / jnp.sum(ex, axis=1, keepdims=True)
    probs_ref[...] = probs

    eidx = jax.lax.broadcasted_iota(jnp.int32, (BB, NUM_EXPERTS), 1)
    m1 = jnp.max(probs, axis=1, keepdims=True)
    i1 = jnp.min(jnp.where(probs >= m1, eidx, NUM_EXPERTS), axis=1, keepdims=True)
    p_rest = jnp.where(eidx == i1, -1.0, probs)
    m2 = jnp.max(p_rest, axis=1, keepdims=True)
    i2 = jnp.min(jnp.where(p_rest >= m2, eidx, NUM_EXPERTS), axis=1, keepdims=True)
    denom = m1 + m2
    w1 = m1 / denom
    w2 = m2 / denom

    all_logits = jnp.dot(cls, wall_ref[...], preferred_element_type=jnp.float32) + ball_ref[...]

    jidx = jax.lax.broadcasted_iota(jnp.int32, (BB, EL), 1)
    e_of_j = jidx // EXPERT_LABELS
    sel1 = e_of_j == i1
    sel2 = e_of_j == i2
    expert_ref[...] = jnp.where(sel1 | sel2, all_logits, 0.0).reshape(
        BB, NUM_EXPERTS, EXPERT_LABELS)

    wlane = jnp.where(sel1, w1, jnp.where(sel2, w2, 0.0))
    contrib = all_logits * wlane
    even = (jidx % EXPERT_LABELS) == 0
    f0 = jnp.sum(jnp.where(even, contrib, 0.0), axis=1, keepdims=True)
    f1 = jnp.sum(jnp.where(even, 0.0, contrib), axis=1, keepdims=True)
    final_ref[...] = jnp.concatenate([f0, f1], axis=1)


@functools.partial(jax.jit, static_argnames=("interpret",))
def kernel(hidden_state, Wg1, bg1, Wg2, bg2, We, be, interpret=False):
    B, S, D = hidden_state.shape
    grid = (B // BB,)

    W_all = We.transpose(1, 0, 2).reshape(D, EL)
    b_all = be.reshape(1, EL)
    bg1r = bg1.reshape(1, D)
    bg2r = bg2.reshape(1, NUM_EXPERTS)

    final, probs, expert = pl.pallas_call(
        _moe_body,
        grid=grid,
        in_specs=[
            pl.BlockSpec(memory_space=pl.ANY),
            pl.BlockSpec((D, D), lambda i: (0, 0)),
            pl.BlockSpec((1, D), lambda i: (0, 0)),
            pl.BlockSpec((D, NUM_EXPERTS), lambda i: (0, 0)),
            pl.BlockSpec((1, NUM_EXPERTS), lambda i: (0, 0)),
            pl.BlockSpec((D, EL), lambda i: (0, 0)),
            pl.BlockSpec((1, EL), lambda i: (0, 0)),
        ],
        out_specs=[
            pl.BlockSpec((BB, EXPERT_LABELS), lambda i: (i, 0)),
            pl.BlockSpec((BB, NUM_EXPERTS), lambda i: (i, 0)),
            pl.BlockSpec((BB, NUM_EXPERTS, EXPERT_LABELS), lambda i: (i, 0, 0)),
        ],
        out_shape=[
            jax.ShapeDtypeStruct((B, EXPERT_LABELS), jnp.float32),
            jax.ShapeDtypeStruct((B, NUM_EXPERTS), jnp.float32),
            jax.ShapeDtypeStruct((B, NUM_EXPERTS, EXPERT_LABELS), jnp.float32),
        ],
        scratch_shapes=[
            pltpu.VMEM((NB, BB, D), jnp.float32),
            pltpu.SemaphoreType.DMA((NB, NSPLIT)),
        ],
        compiler_params=pltpu.CompilerParams(
            dimension_semantics=("arbitrary",),
        ),
        interpret=interpret,
    )(hidden_state, Wg1, bg1r, Wg2, bg2r, W_all, b_all)

    return final, probs, expert


# speedup vs baseline: 1.6042x; 1.6042x over previous
import functools

import jax
import jax.numpy as jnp
from jax.experimental import pallas as pl
from jax.experimental.pallas import tpu as pltpu

NUM_EXPERTS = 5
TOP_K = 2
EXPERT_LABELS = 2
HIDDEN = 768
EL = NUM_EXPERTS * EXPERT_LABELS
BB = 1024
NSPLIT = 4
NB = 4096 // BB


def _moe_body(hs_hbm, wg1_ref, bg1_ref, wg2_ref, bg2_ref, wall_ref, ball_ref,
              final_ref, probs_ref, expert_ref, cls_buf, sems):
    i = pl.program_id(0)
    n = pl.num_programs(0)

    CH = BB // NSPLIT

    def cls_copy(slot, j):
        return pltpu.make_async_copy(
            hs_hbm.at[pl.ds(slot * BB + j * CH, CH), 0, :],
            cls_buf.at[slot, pl.ds(j * CH, CH), :],
            sems.at[slot, j],
        )

    @pl.when(i == 0)
    def _prime():
        for s in range(NB):
            for j in range(NSPLIT):
                cls_copy(s, j).start()

    for j in range(NSPLIT):
        cls_copy(i, j).wait()
    cls = cls_buf[i]

    h = jnp.maximum(
        jnp.dot(cls, wg1_ref[...], preferred_element_type=jnp.float32)
        + bg1_ref[...], 0.0)
    glog = jnp.dot(h, wg2_ref[...], preferred_element_type=jnp.float32) + bg2_ref[...]

    ek = jax.lax.broadcasted_iota(jnp.int32, (NUM_EXPERTS, NUM_EXPERTS), 0)
    ej = jax.lax.broadcasted_iota(jnp.int32, (NUM_EXPERTS, NUM_EXPERTS), 1)
    ones55 = jnp.ones((NUM_EXPERTS, NUM_EXPERTS), jnp.float32)
    lt55 = (ek <= ej).astype(jnp.float32)
    re = jax.lax.broadcasted_iota(jnp.int32, (NUM_EXPERTS, EL), 0)
    rj = jax.lax.broadcasted_iota(jnp.int32, (NUM_EXPERTS, EL), 1)
    rep = (rj // EXPERT_LABELS == re).astype(jnp.float32)
    pj = jax.lax.broadcasted_iota(jnp.int32, (EL, EXPERT_LABELS), 0)
    pl_ = jax.lax.broadcasted_iota(jnp.int32, (EL, EXPERT_LABELS), 1)
    par = (pj % EXPERT_LABELS == pl_).astype(jnp.float32)

    def rbcast(x, m):
        return jnp.dot(x, m, preferred_element_type=jnp.float32)

    gmax = jnp.max(glog, axis=1, keepdims=True)
    ex = jnp.exp(glog - gmax)
    probs = ex / rbcast(ex, ones55)
    probs_ref[...] = probs

    is1 = (glog >= gmax).astype(jnp.float32)
    sel1 = is1 * (rbcast(is1, lt55) == 1.0).astype(jnp.float32)
    g2 = glog - sel1 * 3.4e38
    g2max = jnp.max(g2, axis=1, keepdims=True)
    is2 = (g2 >= g2max).astype(jnp.float32)
    sel2 = is2 * (rbcast(is2, lt55) == 1.0).astype(jnp.float32)

    p1 = rbcast(sel1 * probs, ones55)
    p2 = rbcast(sel2 * probs, ones55)
    w5 = (sel1 * p1 + sel2 * p2) / (p1 + p2)

    all_logits = jnp.dot(cls, wall_ref[...], preferred_element_type=jnp.float32) + ball_ref[...]

    expert_ref[...] = all_logits * rbcast(sel1 + sel2, rep)
    final_ref[...] = rbcast(all_logits * rbcast(w5, rep), par)


@functools.partial(jax.jit, static_argnames=("interpret",))
def kernel(hidden_state, Wg1, bg1, Wg2, bg2, We, be, interpret=False):
    B, S, D = hidden_state.shape
    grid = (B // BB,)

    W_all = We.transpose(1, 0, 2).reshape(D, EL)
    b_all = be.reshape(1, EL)
    bg1r = bg1.reshape(1, D)
    bg2r = bg2.reshape(1, NUM_EXPERTS)

    final, probs, expert = pl.pallas_call(
        _moe_body,
        grid=grid,
        in_specs=[
            pl.BlockSpec(memory_space=pl.ANY),
            pl.BlockSpec((D, D), lambda i: (0, 0)),
            pl.BlockSpec((1, D), lambda i: (0, 0)),
            pl.BlockSpec((D, NUM_EXPERTS), lambda i: (0, 0)),
            pl.BlockSpec((1, NUM_EXPERTS), lambda i: (0, 0)),
            pl.BlockSpec((D, EL), lambda i: (0, 0)),
            pl.BlockSpec((1, EL), lambda i: (0, 0)),
        ],
        out_specs=[
            pl.BlockSpec((BB, EXPERT_LABELS), lambda i: (i, 0)),
            pl.BlockSpec((BB, NUM_EXPERTS), lambda i: (i, 0)),
            pl.BlockSpec((BB, EL), lambda i: (i, 0)),
        ],
        out_shape=[
            jax.ShapeDtypeStruct((B, EXPERT_LABELS), jnp.float32),
            jax.ShapeDtypeStruct((B, NUM_EXPERTS), jnp.float32),
            jax.ShapeDtypeStruct((B, EL), jnp.float32),
        ],
        scratch_shapes=[
            pltpu.VMEM((NB, BB, D), jnp.float32),
            pltpu.SemaphoreType.DMA((NB, NSPLIT)),
        ],
        compiler_params=pltpu.CompilerParams(
            dimension_semantics=("arbitrary",),
        ),
        interpret=interpret,
    )(hidden_state, Wg1, bg1r, Wg2, bg2r, W_all, b_all)

    return final, probs, expert.reshape(B, NUM_EXPERTS, EXPERT_LABELS)
